# baseline (speedup 1.0000x reference)
import jax
import jax.numpy as jnp
from jax.experimental import pallas as pl
from jax.experimental.pallas import tpu as pltpu

_VMEM_LIMIT = 56 * 1024 * 1024
_BLOCK_N = 4


def _pad_hw1(y):
    B, H, W, C = y.shape
    zw = jnp.zeros((B, H, 1, C), y.dtype)
    t = jnp.concatenate([zw, y, zw], axis=2)
    zh = jnp.zeros((B, 1, W + 2, C), y.dtype)
    return jnp.concatenate([zh, t, zh], axis=1)


def _taps(w_ref):
    w = w_ref[...].astype(jnp.bfloat16)
    return w.reshape(w.shape[0] * w.shape[1], w.shape[2])


def _pair(y4, wa_ref, ba_ref, wb_ref, bb_ref):
    B, H, W, C = y4.shape
    M = B * H * W
    yp_ = _pad_hw1(y4)
    la = jnp.concatenate(
        [yp_[:, 1:1 + H, k:k + W, :].reshape(M, C) for k in range(3)], axis=1)
    a = jnp.maximum(
        jnp.dot(la, _taps(wa_ref), preferred_element_type=jnp.float32)
        + ba_ref[0], 0.0)
    lb = jnp.concatenate(
        [yp_[:, k:k + H, 1:1 + W, :].reshape(M, C) for k in range(3)], axis=1)
    b = jnp.maximum(
        jnp.dot(lb, _taps(wb_ref), preferred_element_type=jnp.float32)
        + bb_ref[0], 0.0)
    return a, b


def _fused_kernel(x_ref,
                  w3_ref, w1t_ref, wdt_ref, wpt_ref,
                  b1_ref, b3_ref, ba1_ref, bb1_ref, bd_ref, b2_ref,
                  ba2_ref, bb2_ref, bp_ref,
                  wa1_ref, wb1_ref, w2_ref, wa2_ref, wb2_ref,
                  o_ref):
    B, H, W, Cin = x_ref.shape
    M = B * H * W
    cp = wpt_ref.shape[0]
    cd = wdt_ref.shape[0]
    c2 = w2_ref.shape[2]

    x = x_ref[...].reshape(M, Cin).astype(jnp.bfloat16)

    def _stem_t(wt_ref):
        return jax.lax.dot_general(
            x, wt_ref[...].astype(jnp.bfloat16),
            dimension_numbers=(((1,), (1,)), ((), ())),
            preferred_element_type=jnp.float32)

    y3 = jnp.maximum(
        jnp.dot(x, w3_ref[...].astype(jnp.bfloat16),
                preferred_element_type=jnp.float32) + b3_ref[0], 0.0)
    y1 = jnp.maximum(_stem_t(w1t_ref) + b1_ref[0], 0.0)
    yd = jnp.maximum(_stem_t(wdt_ref) + bd_ref[0], 0.0)
    zp = _stem_t(wpt_ref)

    zpp = _pad_hw1(zp.reshape(B, H, W, cp))
    hs = zpp[:, :, 0:W, :] + zpp[:, :, 1:W + 1, :] + zpp[:, :, 2:W + 2, :]
    vs = hs[:, 0:H] + hs[:, 1:H + 1] + hs[:, 2:H + 2]
    yp = jnp.maximum(vs.reshape(M, cp) * (1.0 / 9.0) + bp_ref[0], 0.0)

    c3 = y3.shape[1]
    a1, b1_ = _pair(y3.astype(jnp.bfloat16).reshape(B, H, W, c3),
                    wa1_ref, ba1_ref, wb1_ref, bb1_ref)

    ydp = _pad_hw1(yd.astype(jnp.bfloat16).reshape(B, H, W, cd))
    l2 = jnp.concatenate(
        [ydp[:, kh:kh + H, kw:kw + W, :].reshape(M, cd)
         for kh in range(3) for kw in range(3)], axis=1)
    t = jnp.maximum(
        jnp.dot(l2, _taps(w2_ref), preferred_element_type=jnp.float32)
        + b2_ref[0], 0.0)
    a2, b2_ = _pair(t.astype(jnp.bfloat16).reshape(B, H, W, c2),
                    wa2_ref, ba2_ref, wb2_ref, bb2_ref)

    out = jnp.concatenate([y1, a1, b1_, a2, b2_, yp], axis=1)
    o_ref[...] = out.reshape(B, H, W, out.shape[1]).astype(o_ref.dtype)


def kernel(x, branch1x1_w, branch1x1_b, branch3x3_1_w, branch3x3_1_b,
           branch3x3_2a_w, branch3x3_2a_b, branch3x3_2b_w, branch3x3_2b_b,
           branch3x3dbl_1_w, branch3x3dbl_1_b, branch3x3dbl_2_w,
           branch3x3dbl_2_b, branch3x3dbl_3a_w, branch3x3dbl_3a_b,
           branch3x3dbl_3b_w, branch3x3dbl_3b_b, branch_pool_w,
           branch_pool_b):
    N, Cin, H, W = x.shape
    xh = jnp.transpose(x, (0, 2, 3, 1))

    cout = (branch1x1_w.shape[1] + 4 * branch3x3_2a_w.shape[2]
            + branch_pool_w.shape[1])
    B = _BLOCK_N

    args = [branch3x3_1_w, branch1x1_w.T, branch3x3dbl_1_w.T,
            branch_pool_w.T,
            branch1x1_b, branch3x3_1_b, branch3x3_2a_b, branch3x3_2b_b,
            branch3x3dbl_1_b, branch3x3dbl_2_b, branch3x3dbl_3a_b,
            branch3x3dbl_3b_b, branch_pool_b,
            branch3x3_2a_w, branch3x3_2b_w, branch3x3dbl_2_w,
            branch3x3dbl_3a_w, branch3x3dbl_3b_w]

    def bspec(a):
        return pl.BlockSpec(a.shape, lambda n: (0,) * a.ndim)

    out = pl.pallas_call(
        _fused_kernel,
        out_shape=jax.ShapeDtypeStruct((N, H, W, cout), x.dtype),
        grid=(N // B,),
        in_specs=[pl.BlockSpec((B, H, W, Cin), lambda n: (n, 0, 0, 0))]
        + [bspec(a) for a in args],
        out_specs=pl.BlockSpec((B, H, W, cout), lambda n: (n, 0, 0, 0)),
        compiler_params=pltpu.CompilerParams(
            dimension_semantics=("parallel",),
            vmem_limit_bytes=_VMEM_LIMIT),
    )(xh, *args)

    return jnp.transpose(out, (0, 3, 1, 2))


# speedup vs baseline: 1.0489x; 1.0489x over previous
import jax
import jax.numpy as jnp
from jax.experimental import pallas as pl
from jax.experimental.pallas import tpu as pltpu

_VMEM_LIMIT = 56 * 1024 * 1024
_BLOCK_N = 8


def _taps(w_ref):
    w = w_ref[...].astype(jnp.bfloat16)
    return w.reshape(w.shape[0] * w.shape[1], w.shape[2])


def _shift_tap(yf, dh, dw, H, W):
    M = yf.shape[0]
    s = dh * W + dw
    r = jnp.roll(yf, -s, axis=0) if s else yf
    mi = jax.lax.broadcasted_iota(jnp.int32, (M, 1), 0)
    cond = None
    if dw:
        wv = jax.lax.rem(mi, W)
        cond = (wv >= -dw) if dw < 0 else (wv < W - dw)
    if dh:
        hv = jax.lax.rem(mi // W, H)
        c2 = (hv >= -dh) if dh < 0 else (hv < H - dh)
        cond = c2 if cond is None else (cond & c2)
    if cond is None:
        return r
    return jnp.where(cond, r, jnp.zeros((), r.dtype))


def _pair(yf, H, W, wa_ref, ba_ref, wb_ref, bb_ref):
    la = jnp.concatenate(
        [_shift_tap(yf, 0, k - 1, H, W) for k in range(3)], axis=1)
    a = jnp.maximum(
        jnp.dot(la, _taps(wa_ref), preferred_element_type=jnp.float32)
        + ba_ref[0], 0.0)
    lb = jnp.concatenate(
        [_shift_tap(yf, k - 1, 0, H, W) for k in range(3)], axis=1)
    b = jnp.maximum(
        jnp.dot(lb, _taps(wb_ref), preferred_element_type=jnp.float32)
        + bb_ref[0], 0.0)
    return a, b


def _fused_kernel(x_ref,
                  w3_ref, w1t_ref, wdt_ref, wpt_ref,
                  b1_ref, b3_ref, ba1_ref, bb1_ref, bd_ref, b2_ref,
                  ba2_ref, bb2_ref, bp_ref,
                  wa1_ref, wb1_ref, w2_ref, wa2_ref, wb2_ref,
                  o_ref):
    B, H, W, Cin = x_ref.shape
    M = B * H * W

    x = x_ref[...].reshape(M, Cin).astype(jnp.bfloat16)

    def _stem_t(wt_ref):
        return jax.lax.dot_general(
            x, wt_ref[...].astype(jnp.bfloat16),
            dimension_numbers=(((1,), (1,)), ((), ())),
            preferred_element_type=jnp.float32)

    y3 = jnp.maximum(
        jnp.dot(x, w3_ref[...].astype(jnp.bfloat16),
                preferred_element_type=jnp.float32) + b3_ref[0], 0.0)
    y1 = jnp.maximum(_stem_t(w1t_ref) + b1_ref[0], 0.0)
    yd = jnp.maximum(_stem_t(wdt_ref) + bd_ref[0], 0.0)
    zp = _stem_t(wpt_ref)

    hsum = (_shift_tap(zp, 0, -1, H, W) + zp + _shift_tap(zp, 0, 1, H, W))
    vs = (_shift_tap(hsum, -1, 0, H, W) + hsum
          + _shift_tap(hsum, 1, 0, H, W))
    yp = jnp.maximum(vs * (1.0 / 9.0) + bp_ref[0], 0.0)

    a1, b1_ = _pair(y3.astype(jnp.bfloat16), H, W,
                    wa1_ref, ba1_ref, wb1_ref, bb1_ref)

    ydb = yd.astype(jnp.bfloat16)
    l2 = jnp.concatenate(
        [_shift_tap(ydb, kh - 1, kw - 1, H, W)
         for kh in range(3) for kw in range(3)], axis=1)
    t = jnp.maximum(
        jnp.dot(l2, _taps(w2_ref), preferred_element_type=jnp.float32)
        + b2_ref[0], 0.0)
    a2, b2_ = _pair(t.astype(jnp.bfloat16), H, W,
                    wa2_ref, ba2_ref, wb2_ref, bb2_ref)

    out = jnp.concatenate([y1, a1, b1_, a2, b2_, yp], axis=1)
    o_ref[...] = out.reshape(B, H, W, out.shape[1]).astype(o_ref.dtype)


def kernel(x, branch1x1_w, branch1x1_b, branch3x3_1_w, branch3x3_1_b,
           branch3x3_2a_w, branch3x3_2a_b, branch3x3_2b_w, branch3x3_2b_b,
           branch3x3dbl_1_w, branch3x3dbl_1_b, branch3x3dbl_2_w,
           branch3x3dbl_2_b, branch3x3dbl_3a_w, branch3x3dbl_3a_b,
           branch3x3dbl_3b_w, branch3x3dbl_3b_b, branch_pool_w,
           branch_pool_b):
    N, Cin, H, W = x.shape
    xh = jnp.transpose(x, (0, 2, 3, 1))

    cout = (branch1x1_w.shape[1] + 4 * branch3x3_2a_w.shape[2]
            + branch_pool_w.shape[1])
    B = _BLOCK_N

    args = [branch3x3_1_w, branch1x1_w.T, branch3x3dbl_1_w.T,
            branch_pool_w.T,
            branch1x1_b, branch3x3_1_b, branch3x3_2a_b, branch3x3_2b_b,
            branch3x3dbl_1_b, branch3x3dbl_2_b, branch3x3dbl_3a_b,
            branch3x3dbl_3b_b, branch_pool_b,
            branch3x3_2a_w, branch3x3_2b_w, branch3x3dbl_2_w,
            branch3x3dbl_3a_w, branch3x3dbl_3b_w]

    def bspec(a):
        return pl.BlockSpec(a.shape, lambda n: (0,) * a.ndim)

    out = pl.pallas_call(
        _fused_kernel,
        out_shape=jax.ShapeDtypeStruct((N, H, W, cout), x.dtype),
        grid=(N // B,),
        in_specs=[pl.BlockSpec((B, H, W, Cin), lambda n: (n, 0, 0, 0))]
        + [bspec(a) for a in args],
        out_specs=pl.BlockSpec((B, H, W, cout), lambda n: (n, 0, 0, 0)),
        compiler_params=pltpu.CompilerParams(
            dimension_semantics=("parallel",),
            vmem_limit_bytes=_VMEM_LIMIT),
    )(xh, *args)

    return jnp.transpose(out, (0, 3, 1, 2))
